# baseline (speedup 1.0000x reference)
import functools

import jax
import jax.numpy as jnp
from jax import lax
from jax.experimental import pallas as pl
from jax.experimental.pallas import tpu as pltpu
from jax.experimental.pallas import tpu_sc as plsc

N = 10000
E = 320000
IN_DIM = 128
HID = 64
OUT_DIM = 128
G = 64

NC = 2
NS = 16
NW = NC * NS
GRP = 128
WG = 80
EP = NW * WG * GRP
TG = EP // GRP
KG = 4
STEPS = WG // KG
R = 10240
ZROWS = R // NS

_sc_mesh = plsc.VectorSubcoreMesh(core_axis_name="c", subcore_axis_name="s")


@functools.partial(
    pl.kernel,
    out_type=jax.ShapeDtypeStruct((NC * R, HID), jnp.float32),
    mesh=_sc_mesh,
    scratch_types=[
        pltpu.VMEM((STEPS, KG * GRP), jnp.int32),
        pltpu.VMEM((STEPS, KG * GRP), jnp.int32),
        pltpu.VMEM((KG * GRP, HID), jnp.float32),
        pltpu.VMEM((KG * GRP, HID), jnp.float32),
        pltpu.VMEM_SHARED((R, HID), jnp.float32),
        pltpu.SemaphoreType.DMA,
        pltpu.SemaphoreType.DMA,
        pltpu.SemaphoreType.DMA,
        pltpu.SemaphoreType.DMA,
    ],
    compiler_params=pltpu.CompilerParams(use_tc_tiling_on_sc=False),
)
def _sc_agg(h_hbm, src_hbm, dst_hbm, z_hbm, out_hbm,
            src_v, dst_v, rows0, rows1, agg_sh, gs0, gs1, ss0, ss1):
    cid = lax.axis_index("c")
    sid = lax.axis_index("s")
    wid = sid * NC + cid
    pltpu.sync_copy(z_hbm, agg_sh.at[pl.ds(sid * ZROWS, ZROWS)])
    pltpu.sync_copy(src_hbm.at[wid], src_v)
    pltpu.sync_copy(dst_hbm.at[wid], dst_v)
    plsc.subcore_barrier()

    pltpu.async_copy(h_hbm.at[src_v.at[0]], rows0, gs0)

    @pl.loop(0, STEPS, step=2)
    def _(j):
        pltpu.make_async_copy(h_hbm.at[src_v.at[j]], rows0, gs0).wait()

        @pl.when(j >= 1)
        def _():
            pltpu.make_async_copy(
                rows1, agg_sh.at[dst_v.at[j - 1]], ss1).wait()

        pltpu.async_copy(h_hbm.at[src_v.at[j + 1]], rows1, gs1)
        pltpu.async_copy(rows0, agg_sh.at[dst_v.at[j]], ss0, add=True)

        pltpu.make_async_copy(h_hbm.at[src_v.at[j + 1]], rows1, gs1).wait()

        @pl.when(j + 2 < STEPS)
        def _():
            pltpu.make_async_copy(
                rows0, agg_sh.at[dst_v.at[j]], ss0).wait()
            pltpu.async_copy(h_hbm.at[src_v.at[j + 2]], rows0, gs0)

        pltpu.async_copy(rows1, agg_sh.at[dst_v.at[j + 1]], ss1, add=True)

    pltpu.make_async_copy(rows0, agg_sh.at[dst_v.at[STEPS - 2]], ss0).wait()
    pltpu.make_async_copy(rows1, agg_sh.at[dst_v.at[STEPS - 1]], ss1).wait()

    plsc.subcore_barrier()
    pltpu.sync_copy(agg_sh.at[pl.ds(sid * ZROWS, ZROWS)],
                    out_hbm.at[pl.ds(cid * R + sid * ZROWS, ZROWS)])


def _proj_in_body(x_ref, w_ref, b_ref, o_ref):
    o_ref[...] = jnp.dot(x_ref[...], w_ref[...],
                         preferred_element_type=jnp.float32) + b_ref[...]


def _gin_mlp_body(h_ref, p0_ref, p1_ref, w1_ref, b1_ref, w2_ref, b2_ref, o_ref):
    m = h_ref[...] + p0_ref[...] + p1_ref[...]
    t = jnp.maximum(jnp.dot(m, w1_ref[...],
                            preferred_element_type=jnp.float32) + b1_ref[...], 0.0)
    o_ref[...] = jnp.maximum(jnp.dot(t, w2_ref[...],
                                     preferred_element_type=jnp.float32) + b2_ref[...], 0.0)


def _final_body(h_ref, p0_ref, p1_ref, w1_ref, b1_ref, w2_ref, b2_ref,
                wo_ref, bo_ref, batch_ref, o_ref):
    m = h_ref[...] + p0_ref[...] + p1_ref[...]
    t = jnp.maximum(jnp.dot(m, w1_ref[...],
                            preferred_element_type=jnp.float32) + b1_ref[...], 0.0)
    h2 = jnp.maximum(jnp.dot(t, w2_ref[...],
                             preferred_element_type=jnp.float32) + b2_ref[...], 0.0)
    ho = jnp.dot(h2, wo_ref[...], preferred_element_type=jnp.float32) + bo_ref[...]
    gids = lax.broadcasted_iota(jnp.int32, (N, G), 1)
    onehot = jnp.where(batch_ref[...] == gids, 1.0, 0.0)
    sums = lax.dot_general(onehot, ho, (((0,), (0,)), ((), ())),
                           preferred_element_type=jnp.float32)
    ones = jnp.ones((N, 1), jnp.float32)
    counts = lax.dot_general(onehot, ones, (((0,), (0,)), ((), ())),
                             preferred_element_type=jnp.float32)
    o_ref[...] = sums / jnp.maximum(counts, 1.0)


def kernel(x, edge_index, batch, W_in, b_in, W1_0, b1_0, W2_0, b2_0,
           W1_1, b1_1, W2_1, b2_1, W_out, b_out):
    pad = EP - E
    src2d = jnp.concatenate(
        [edge_index[0], jnp.zeros((pad,), jnp.int32)]).reshape(NW, STEPS, KG * GRP)
    dst2d = jnp.concatenate(
        [edge_index[1], jnp.full((pad,), N, jnp.int32)]).reshape(NW, STEPS, KG * GRP)
    zeros_blk = jnp.zeros((ZROWS, HID), jnp.float32)
    batch2d = batch.reshape(N, 1)
    b_in2 = b_in.reshape(1, HID)
    b1_0r, b2_0r = b1_0.reshape(1, HID), b2_0.reshape(1, HID)
    b1_1r, b2_1r = b1_1.reshape(1, HID), b2_1.reshape(1, HID)
    b_out2 = b_out.reshape(1, OUT_DIM)

    h = pl.pallas_call(
        _proj_in_body,
        out_shape=jax.ShapeDtypeStruct((N, HID), jnp.float32),
    )(x, W_in, b_in2)

    p = _sc_agg(h, src2d, dst2d, zeros_blk)

    h = pl.pallas_call(
        _gin_mlp_body,
        out_shape=jax.ShapeDtypeStruct((N, HID), jnp.float32),
    )(h, p[:N], p[R:R + N], W1_0, b1_0r, W2_0, b2_0r)

    p = _sc_agg(h, src2d, dst2d, zeros_blk)

    out = pl.pallas_call(
        _final_body,
        out_shape=jax.ShapeDtypeStruct((G, OUT_DIM), jnp.float32),
    )(h, p[:N], p[R:R + N], W1_1, b1_1r, W2_1, b2_1r, W_out, b_out2, batch2d)
    return out


# speedup vs baseline: 2.1406x; 2.1406x over previous
import functools

import jax
import jax.numpy as jnp
from jax import lax
from jax.experimental import pallas as pl
from jax.experimental.pallas import tpu as pltpu
from jax.experimental.pallas import tpu_sc as plsc

N = 10000
E = 320000
IN_DIM = 128
HID = 64
OUT_DIM = 128
G = 64

NC = 2
NS = 16
NW = NC * NS
GRP = 128
WG = 80
EP = NW * WG * GRP
TG = EP // GRP
KG = 4
EPW = EP // NS
STEPS = EPW // (KG * GRP)
HH = HID // NC
R = 10240
ZROWS = R // NS
HROWS = N // NS

_sc_mesh = plsc.VectorSubcoreMesh(core_axis_name="c", subcore_axis_name="s")


@functools.partial(
    pl.kernel,
    out_type=jax.ShapeDtypeStruct((NC * R, HH), jnp.float32),
    mesh=_sc_mesh,
    scratch_types=[
        pltpu.VMEM((STEPS, KG * GRP), jnp.int32),
        pltpu.VMEM((STEPS, KG * GRP), jnp.int32),
        pltpu.VMEM((KG * GRP, HH), jnp.float32),
        pltpu.VMEM((KG * GRP, HH), jnp.float32),
        pltpu.VMEM_SHARED((R, HH), jnp.float32),
        pltpu.VMEM_SHARED((N, HH), jnp.float32),
        pltpu.SemaphoreType.DMA,
        pltpu.SemaphoreType.DMA,
        pltpu.SemaphoreType.DMA,
        pltpu.SemaphoreType.DMA,
    ],
    compiler_params=pltpu.CompilerParams(use_tc_tiling_on_sc=False),
)
def _sc_agg(h_hbm, src_hbm, dst_hbm, z_hbm, out_hbm,
            src_v, dst_v, rows0, rows1, agg_sh, h_sh, gs0, gs1, ss0, ss1):
    cid = lax.axis_index("c")
    sid = lax.axis_index("s")
    pltpu.sync_copy(z_hbm, agg_sh.at[pl.ds(sid * ZROWS, ZROWS)])
    pltpu.sync_copy(h_hbm.at[cid].at[pl.ds(sid * HROWS, HROWS)],
                    h_sh.at[pl.ds(sid * HROWS, HROWS)])
    pltpu.sync_copy(src_hbm.at[sid], src_v)
    pltpu.sync_copy(dst_hbm.at[sid], dst_v)
    plsc.subcore_barrier()

    pltpu.async_copy(h_sh.at[src_v.at[0]], rows0, gs0)

    @pl.loop(0, STEPS, step=2)
    def _(j):
        pltpu.make_async_copy(h_sh.at[src_v.at[j]], rows0, gs0).wait()

        @pl.when(j >= 1)
        def _():
            pltpu.make_async_copy(
                rows1, agg_sh.at[dst_v.at[j - 1]], ss1).wait()

        pltpu.async_copy(h_sh.at[src_v.at[j + 1]], rows1, gs1)
        pltpu.async_copy(rows0, agg_sh.at[dst_v.at[j]], ss0, add=True)

        pltpu.make_async_copy(h_sh.at[src_v.at[j + 1]], rows1, gs1).wait()

        @pl.when(j + 2 < STEPS)
        def _():
            pltpu.make_async_copy(
                rows0, agg_sh.at[dst_v.at[j]], ss0).wait()
            pltpu.async_copy(h_sh.at[src_v.at[j + 2]], rows0, gs0)

        pltpu.async_copy(rows1, agg_sh.at[dst_v.at[j + 1]], ss1, add=True)

    pltpu.make_async_copy(rows0, agg_sh.at[dst_v.at[STEPS - 2]], ss0).wait()
    pltpu.make_async_copy(rows1, agg_sh.at[dst_v.at[STEPS - 1]], ss1).wait()

    plsc.subcore_barrier()
    pltpu.sync_copy(agg_sh.at[pl.ds(sid * ZROWS, ZROWS)],
                    out_hbm.at[pl.ds(cid * R + sid * ZROWS, ZROWS)])


def _proj_in_body(x_ref, w_ref, b_ref, o_ref):
    o_ref[...] = jnp.dot(x_ref[...], w_ref[...],
                         preferred_element_type=jnp.float32) + b_ref[...]


def _gin_mlp_body(h_ref, p0_ref, p1_ref, w1_ref, b1_ref, w2_ref, b2_ref, o_ref):
    m = h_ref[...] + jnp.concatenate([p0_ref[...], p1_ref[...]], axis=1)
    t = jnp.maximum(jnp.dot(m, w1_ref[...],
                            preferred_element_type=jnp.float32) + b1_ref[...], 0.0)
    o_ref[...] = jnp.maximum(jnp.dot(t, w2_ref[...],
                                     preferred_element_type=jnp.float32) + b2_ref[...], 0.0)


def _final_body(h_ref, p0_ref, p1_ref, w1_ref, b1_ref, w2_ref, b2_ref,
                wo_ref, bo_ref, batch_ref, o_ref):
    m = h_ref[...] + jnp.concatenate([p0_ref[...], p1_ref[...]], axis=1)
    t = jnp.maximum(jnp.dot(m, w1_ref[...],
                            preferred_element_type=jnp.float32) + b1_ref[...], 0.0)
    h2 = jnp.maximum(jnp.dot(t, w2_ref[...],
                             preferred_element_type=jnp.float32) + b2_ref[...], 0.0)
    ho = jnp.dot(h2, wo_ref[...], preferred_element_type=jnp.float32) + bo_ref[...]
    gids = lax.broadcasted_iota(jnp.int32, (N, G), 1)
    onehot = jnp.where(batch_ref[...] == gids, 1.0, 0.0)
    sums = lax.dot_general(onehot, ho, (((0,), (0,)), ((), ())),
                           preferred_element_type=jnp.float32)
    ones = jnp.ones((N, 1), jnp.float32)
    counts = lax.dot_general(onehot, ones, (((0,), (0,)), ((), ())),
                             preferred_element_type=jnp.float32)
    o_ref[...] = sums / jnp.maximum(counts, 1.0)


def kernel(x, edge_index, batch, W_in, b_in, W1_0, b1_0, W2_0, b2_0,
           W1_1, b1_1, W2_1, b2_1, W_out, b_out):
    pad = EP - E
    src2d = jnp.concatenate(
        [edge_index[0], jnp.zeros((pad,), jnp.int32)]).reshape(NS, STEPS, KG * GRP)
    dst2d = jnp.concatenate(
        [edge_index[1], jnp.full((pad,), N, jnp.int32)]).reshape(NS, STEPS, KG * GRP)
    zeros_blk = jnp.zeros((ZROWS, HH), jnp.float32)
    batch2d = batch.reshape(N, 1)
    b_in2 = b_in.reshape(1, HID)
    b1_0r, b2_0r = b1_0.reshape(1, HID), b2_0.reshape(1, HID)
    b1_1r, b2_1r = b1_1.reshape(1, HID), b2_1.reshape(1, HID)
    b_out2 = b_out.reshape(1, OUT_DIM)

    h = pl.pallas_call(
        _proj_in_body,
        out_shape=jax.ShapeDtypeStruct((N, HID), jnp.float32),
    )(x, W_in, b_in2)

    hs = jnp.stack([h[:, :HH], h[:, HH:]])
    p = _sc_agg(hs, src2d, dst2d, zeros_blk)

    h = pl.pallas_call(
        _gin_mlp_body,
        out_shape=jax.ShapeDtypeStruct((N, HID), jnp.float32),
    )(h, p[:N], p[R:R + N], W1_0, b1_0r, W2_0, b2_0r)

    hs = jnp.stack([h[:, :HH], h[:, HH:]])
    p = _sc_agg(hs, src2d, dst2d, zeros_blk)

    out = pl.pallas_call(
        _final_body,
        out_shape=jax.ShapeDtypeStruct((G, OUT_DIM), jnp.float32),
    )(h, p[:N], p[R:R + N], W1_1, b1_1r, W2_1, b2_1r, W_out, b_out2, batch2d)
    return out


# speedup vs baseline: 2.3025x; 1.0756x over previous
import functools

import jax
import jax.numpy as jnp
from jax import lax
from jax.experimental import pallas as pl
from jax.experimental.pallas import tpu as pltpu
from jax.experimental.pallas import tpu_sc as plsc

N = 10000
E = 320000
IN_DIM = 128
HID = 64
OUT_DIM = 128
G = 64

NC = 2
NS = 16
NW = NC * NS
GRP = 128
WG = 80
EP = NW * WG * GRP
TG = EP // GRP
KG = 4
EPW = EP // NS
STEPS = EPW // (KG * GRP)
HH = HID // NC
R = 10240
ZROWS = R // NS
HROWS = N // NS

_sc_mesh = plsc.VectorSubcoreMesh(core_axis_name="c", subcore_axis_name="s")


@functools.partial(
    pl.kernel,
    out_type=jax.ShapeDtypeStruct((NC * R, HH), jnp.float32),
    mesh=_sc_mesh,
    scratch_types=[
        pltpu.VMEM((STEPS, KG * GRP), jnp.int32),
        pltpu.VMEM((STEPS, KG * GRP), jnp.int32),
        pltpu.VMEM((KG * GRP, HH), jnp.float32),
        pltpu.VMEM((KG * GRP, HH), jnp.float32),
        pltpu.VMEM_SHARED((R, HH), jnp.float32),
        pltpu.VMEM_SHARED((N, HH), jnp.float32),
        pltpu.SemaphoreType.DMA,
        pltpu.SemaphoreType.DMA,
        pltpu.SemaphoreType.DMA,
        pltpu.SemaphoreType.DMA,
    ],
    compiler_params=pltpu.CompilerParams(use_tc_tiling_on_sc=False),
)
def _sc_agg(h_hbm, src_hbm, dst_hbm, z_hbm, out_hbm,
            src_v, dst_v, rows0, rows1, agg_sh, h_sh, gs0, gs1, ss0, ss1):
    cid = lax.axis_index("c")
    sid = lax.axis_index("s")
    pltpu.sync_copy(z_hbm, agg_sh.at[pl.ds(sid * ZROWS, ZROWS)])
    pltpu.sync_copy(h_hbm.at[cid].at[pl.ds(sid * HROWS, HROWS)],
                    h_sh.at[pl.ds(sid * HROWS, HROWS)])
    pltpu.sync_copy(src_hbm.at[sid], src_v)
    pltpu.sync_copy(dst_hbm.at[sid], dst_v)
    plsc.subcore_barrier()

    pltpu.async_copy(h_sh.at[src_v.at[0]], rows0, gs0)

    @pl.loop(0, STEPS, step=2)
    def _(j):
        pltpu.make_async_copy(h_sh.at[src_v.at[j]], rows0, gs0).wait()

        @pl.when(j >= 1)
        def _():
            pltpu.make_async_copy(
                rows1, agg_sh.at[dst_v.at[j - 1]], ss1).wait()

        pltpu.async_copy(h_sh.at[src_v.at[j + 1]], rows1, gs1)
        pltpu.async_copy(rows0, agg_sh.at[dst_v.at[j]], ss0, add=True)

        pltpu.make_async_copy(h_sh.at[src_v.at[j + 1]], rows1, gs1).wait()

        @pl.when(j + 2 < STEPS)
        def _():
            pltpu.make_async_copy(
                rows0, agg_sh.at[dst_v.at[j]], ss0).wait()
            pltpu.async_copy(h_sh.at[src_v.at[j + 2]], rows0, gs0)

        pltpu.async_copy(rows1, agg_sh.at[dst_v.at[j + 1]], ss1, add=True)

    pltpu.make_async_copy(rows0, agg_sh.at[dst_v.at[STEPS - 2]], ss0).wait()
    pltpu.make_async_copy(rows1, agg_sh.at[dst_v.at[STEPS - 1]], ss1).wait()

    plsc.subcore_barrier()
    pltpu.sync_copy(agg_sh.at[pl.ds(sid * ZROWS, ZROWS)],
                    out_hbm.at[pl.ds(cid * R + sid * ZROWS, ZROWS)])


def _proj_in_body(x_ref, w_ref, b_ref, o_ref):
    r = jnp.dot(x_ref[...], w_ref[...],
                preferred_element_type=jnp.float32) + b_ref[...]
    o_ref[0] = r[:, :HH]
    o_ref[1] = r[:, HH:]


def _gin_mlp_body(hs_ref, p_ref, w1_ref, b1_ref, w2_ref, b2_ref, o_ref):
    h = jnp.concatenate([hs_ref[0], hs_ref[1]], axis=1)
    pa = p_ref[...]
    m = h + jnp.concatenate([pa[:N], pa[R:R + N]], axis=1)
    t = jnp.maximum(jnp.dot(m, w1_ref[...],
                            preferred_element_type=jnp.float32) + b1_ref[...], 0.0)
    hn = jnp.maximum(jnp.dot(t, w2_ref[...],
                             preferred_element_type=jnp.float32) + b2_ref[...], 0.0)
    o_ref[0] = hn[:, :HH]
    o_ref[1] = hn[:, HH:]


def _final_body(hs_ref, p_ref, w1_ref, b1_ref, w2_ref, b2_ref,
                wo_ref, bo_ref, batch_ref, o_ref):
    h = jnp.concatenate([hs_ref[0], hs_ref[1]], axis=1)
    pa = p_ref[...]
    m = h + jnp.concatenate([pa[:N], pa[R:R + N]], axis=1)
    t = jnp.maximum(jnp.dot(m, w1_ref[...],
                            preferred_element_type=jnp.float32) + b1_ref[...], 0.0)
    h2 = jnp.maximum(jnp.dot(t, w2_ref[...],
                             preferred_element_type=jnp.float32) + b2_ref[...], 0.0)
    ho = jnp.dot(h2, wo_ref[...], preferred_element_type=jnp.float32) + bo_ref[...]
    gids = lax.broadcasted_iota(jnp.int32, (N, G), 1)
    onehot = jnp.where(batch_ref[...] == gids, 1.0, 0.0)
    sums = lax.dot_general(onehot, ho, (((0,), (0,)), ((), ())),
                           preferred_element_type=jnp.float32)
    ones = jnp.ones((N, 1), jnp.float32)
    counts = lax.dot_general(onehot, ones, (((0,), (0,)), ((), ())),
                             preferred_element_type=jnp.float32)
    o_ref[...] = sums / jnp.maximum(counts, 1.0)


def kernel(x, edge_index, batch, W_in, b_in, W1_0, b1_0, W2_0, b2_0,
           W1_1, b1_1, W2_1, b2_1, W_out, b_out):
    pad = EP - E
    src2d = jnp.concatenate(
        [edge_index[0], jnp.zeros((pad,), jnp.int32)]).reshape(NS, STEPS, KG * GRP)
    dst2d = jnp.concatenate(
        [edge_index[1], jnp.full((pad,), N, jnp.int32)]).reshape(NS, STEPS, KG * GRP)
    zeros_blk = jnp.zeros((ZROWS, HH), jnp.float32)
    batch2d = batch.reshape(N, 1)
    b_in2 = b_in.reshape(1, HID)
    b1_0r, b2_0r = b1_0.reshape(1, HID), b2_0.reshape(1, HID)
    b1_1r, b2_1r = b1_1.reshape(1, HID), b2_1.reshape(1, HID)
    b_out2 = b_out.reshape(1, OUT_DIM)

    hs = pl.pallas_call(
        _proj_in_body,
        out_shape=jax.ShapeDtypeStruct((NC, N, HH), jnp.float32),
    )(x, W_in, b_in2)

    p = _sc_agg(hs, src2d, dst2d, zeros_blk)

    hs = pl.pallas_call(
        _gin_mlp_body,
        out_shape=jax.ShapeDtypeStruct((NC, N, HH), jnp.float32),
    )(hs, p, W1_0, b1_0r, W2_0, b2_0r)

    p = _sc_agg(hs, src2d, dst2d, zeros_blk)

    out = pl.pallas_call(
        _final_body,
        out_shape=jax.ShapeDtypeStruct((G, OUT_DIM), jnp.float32),
    )(hs, p, W1_1, b1_1r, W2_1, b2_1r, W_out, b_out2, batch2d)
    return out


# speedup vs baseline: 2.4954x; 1.0838x over previous
import functools

import jax
import jax.numpy as jnp
from jax import lax
from jax.experimental import pallas as pl
from jax.experimental.pallas import tpu as pltpu
from jax.experimental.pallas import tpu_sc as plsc

N = 10000
E = 320000
IN_DIM = 128
HID = 64
OUT_DIM = 128
G = 64

NC = 2
NS = 16
NW = NC * NS
GRP = 128
WG = 80
EP = NW * WG * GRP
TG = EP // GRP
KG = 4
EPW = EP // NS
STEPS = EPW // (KG * GRP)
HH = HID // NC
R = 10240
ZROWS = R // NS
HROWS = N // NS

_sc_mesh = plsc.VectorSubcoreMesh(core_axis_name="c", subcore_axis_name="s")


@functools.partial(
    pl.kernel,
    out_type=jax.ShapeDtypeStruct((R, HID), jnp.float32),
    mesh=_sc_mesh,
    scratch_types=[
        pltpu.VMEM((STEPS, KG * GRP), jnp.int32),
        pltpu.VMEM((STEPS, KG * GRP), jnp.int32),
        pltpu.VMEM((KG * GRP, HH), jnp.float32),
        pltpu.VMEM((KG * GRP, HH), jnp.float32),
        pltpu.VMEM_SHARED((R, HH), jnp.float32),
        pltpu.VMEM_SHARED((N, HH), jnp.float32),
        pltpu.SemaphoreType.DMA,
        pltpu.SemaphoreType.DMA,
        pltpu.SemaphoreType.DMA,
        pltpu.SemaphoreType.DMA,
    ],
    compiler_params=pltpu.CompilerParams(use_tc_tiling_on_sc=False),
)
def _sc_agg(h_hbm, src_hbm, dst_hbm, z_hbm, out_hbm,
            src_v, dst_v, rows0, rows1, agg_sh, h_sh, gs0, gs1, ss0, ss1):
    cid = lax.axis_index("c")
    sid = lax.axis_index("s")
    pltpu.sync_copy(z_hbm, agg_sh.at[pl.ds(sid * ZROWS, ZROWS)])
    pltpu.sync_copy(h_hbm.at[pl.ds(sid * HROWS, HROWS), pl.ds(cid * HH, HH)],
                    h_sh.at[pl.ds(sid * HROWS, HROWS)])
    pltpu.sync_copy(src_hbm.at[sid], src_v)
    pltpu.sync_copy(dst_hbm.at[sid], dst_v)
    plsc.subcore_barrier()

    pltpu.async_copy(h_sh.at[src_v.at[0]], rows0, gs0)

    @pl.loop(0, STEPS, step=2)
    def _(j):
        pltpu.make_async_copy(h_sh.at[src_v.at[j]], rows0, gs0).wait()

        @pl.when(j >= 1)
        def _():
            pltpu.make_async_copy(
                rows1, agg_sh.at[dst_v.at[j - 1]], ss1).wait()

        pltpu.async_copy(h_sh.at[src_v.at[j + 1]], rows1, gs1)
        pltpu.async_copy(rows0, agg_sh.at[dst_v.at[j]], ss0, add=True)

        pltpu.make_async_copy(h_sh.at[src_v.at[j + 1]], rows1, gs1).wait()

        @pl.when(j + 2 < STEPS)
        def _():
            pltpu.make_async_copy(
                rows0, agg_sh.at[dst_v.at[j]], ss0).wait()
            pltpu.async_copy(h_sh.at[src_v.at[j + 2]], rows0, gs0)

        pltpu.async_copy(rows1, agg_sh.at[dst_v.at[j + 1]], ss1, add=True)

    pltpu.make_async_copy(rows0, agg_sh.at[dst_v.at[STEPS - 2]], ss0).wait()
    pltpu.make_async_copy(rows1, agg_sh.at[dst_v.at[STEPS - 1]], ss1).wait()

    plsc.subcore_barrier()
    pltpu.sync_copy(agg_sh.at[pl.ds(sid * ZROWS, ZROWS)],
                    out_hbm.at[pl.ds(sid * ZROWS, ZROWS), pl.ds(cid * HH, HH)])


def _proj_in_body(x_ref, w_ref, b_ref, o_ref):
    o_ref[...] = jnp.dot(x_ref[...], w_ref[...],
                         preferred_element_type=jnp.float32) + b_ref[...]


def _gin_mlp_body(h_ref, p_ref, w1_ref, b1_ref, w2_ref, b2_ref, o_ref):
    m = h_ref[...] + p_ref[:N]
    t = jnp.maximum(jnp.dot(m, w1_ref[...],
                            preferred_element_type=jnp.float32) + b1_ref[...], 0.0)
    o_ref[...] = jnp.maximum(jnp.dot(t, w2_ref[...],
                                     preferred_element_type=jnp.float32) + b2_ref[...], 0.0)


def _final_body(h_ref, p_ref, w1_ref, b1_ref, w2_ref, b2_ref,
                wo_ref, bo_ref, batch_ref, o_ref):
    m = h_ref[...] + p_ref[:N]
    t = jnp.maximum(jnp.dot(m, w1_ref[...],
                            preferred_element_type=jnp.float32) + b1_ref[...], 0.0)
    h2 = jnp.maximum(jnp.dot(t, w2_ref[...],
                             preferred_element_type=jnp.float32) + b2_ref[...], 0.0)
    ho = jnp.dot(h2, wo_ref[...], preferred_element_type=jnp.float32) + bo_ref[...]
    gids = lax.broadcasted_iota(jnp.int32, (N, G), 1)
    onehot = jnp.where(batch_ref[...] == gids, 1.0, 0.0)
    sums = lax.dot_general(onehot, ho, (((0,), (0,)), ((), ())),
                           preferred_element_type=jnp.float32)
    ones = jnp.ones((N, 1), jnp.float32)
    counts = lax.dot_general(onehot, ones, (((0,), (0,)), ((), ())),
                             preferred_element_type=jnp.float32)
    o_ref[...] = sums / jnp.maximum(counts, 1.0)


def kernel(x, edge_index, batch, W_in, b_in, W1_0, b1_0, W2_0, b2_0,
           W1_1, b1_1, W2_1, b2_1, W_out, b_out):
    pad = EP - E
    src2d = jnp.concatenate(
        [edge_index[0], jnp.zeros((pad,), jnp.int32)]).reshape(NS, STEPS, KG * GRP)
    dst2d = jnp.concatenate(
        [edge_index[1], jnp.full((pad,), N, jnp.int32)]).reshape(NS, STEPS, KG * GRP)
    zeros_blk = jnp.zeros((ZROWS, HH), jnp.float32)
    batch2d = batch.reshape(N, 1)
    b_in2 = b_in.reshape(1, HID)
    b1_0r, b2_0r = b1_0.reshape(1, HID), b2_0.reshape(1, HID)
    b1_1r, b2_1r = b1_1.reshape(1, HID), b2_1.reshape(1, HID)
    b_out2 = b_out.reshape(1, OUT_DIM)

    h = pl.pallas_call(
        _proj_in_body,
        out_shape=jax.ShapeDtypeStruct((N, HID), jnp.float32),
    )(x, W_in, b_in2)

    p = _sc_agg(h, src2d, dst2d, zeros_blk)

    h = pl.pallas_call(
        _gin_mlp_body,
        out_shape=jax.ShapeDtypeStruct((N, HID), jnp.float32),
    )(h, p, W1_0, b1_0r, W2_0, b2_0r)

    p = _sc_agg(h, src2d, dst2d, zeros_blk)

    out = pl.pallas_call(
        _final_body,
        out_shape=jax.ShapeDtypeStruct((G, OUT_DIM), jnp.float32),
    )(h, p, W1_1, b1_1r, W2_1, b2_1r, W_out, b_out2, batch2d)
    return out
